# Initial kernel scaffold; baseline (speedup 1.0000x reference)
import jax
import jax.numpy as jnp
from jax.experimental import pallas as pl


def kernel(force_features, edge_vectors, scalar_w1, scalar_b1, scalar_w2, scalar_b2, irrep2_w1, irrep2_b1, irrep2_w2, irrep2_b2, edge_index_dst, batch_idx):
    raise NotImplementedError("write your pallas kernel here")


# speedup vs baseline: 2.5570x; 2.5570x over previous
import functools
import math

import jax
import jax.numpy as jnp
from jax import lax
from jax.experimental import pallas as pl
from jax.experimental.pallas import tpu as pltpu
from jax.experimental.pallas import tpu_sc as plsc

E = 160000
H = 128
NN = 10000
NB = 100

EP = 163840
ROWS = EP // 128
NP = 10016
SENT = 10000

NC, NS = 2, 16
PH1_ROWS = ROWS // NS
PER_TILE = EP // (NC * NS)

BLK = 1600
GRID = E // BLK
NBP = 104


def _change_basis():
    m = [
        [3 ** -0.5, 0, 0, 0, 3 ** -0.5, 0, 0, 0, 3 ** -0.5],
        [0, 0, 0, 0, 0, 2 ** -0.5, 0, -(2 ** -0.5), 0],
        [0, 0, -(2 ** -0.5), 0, 0, 0, 2 ** -0.5, 0, 0],
        [0, 2 ** -0.5, 0, -(2 ** -0.5), 0, 0, 0, 0, 0],
        [0, 0, 0.5 ** 0.5, 0, 0, 0, 0.5 ** 0.5, 0, 0],
        [0, 0, 0, 0, 0, 2 ** -0.5, 0, 2 ** -0.5, 0],
        [-(6 ** -0.5), 0, 0, 0, 2 * 6 ** -0.5, 0, 0, 0, -(6 ** -0.5)],
        [0, 2 ** -0.5, 0, 2 ** -0.5, 0, 0, 0, 0, 0],
        [-(2 ** -0.5), 0, 0, 0, 0, 0, 0, 0, 2 ** -0.5],
    ]
    return jnp.asarray(m, dtype=jnp.float32).T


def _sc_body(dst2d_hbm, dstf_hbm, batch_hbm, w_hbm, g_hbm,
             idx2, idxf, ones, nodew, batchv, wout, gout, counts):
    cid = lax.axis_index("c")
    sid = lax.axis_index("s")
    wid = sid * NC + cid

    def fill16(ref, val, n16):
        def bdy(i, _):
            ref[pl.ds(i * 16, 16)] = jnp.full((16,), val, ref.dtype)
            return 0
        lax.fori_loop(0, n16, bdy, 0)

    fill16(ones, 1.0, 128 // 16)

    @pl.when(sid == 0)
    def _():
        fill16(nodew, 0.0, NP // 16)
        pltpu.sync_copy(nodew, counts)

    plsc.subcore_barrier()

    pltpu.sync_copy(dst2d_hbm.at[pl.ds(sid * PH1_ROWS, PH1_ROWS)], idx2)

    def scat(j, _):
        pltpu.sync_copy(ones, counts.at[idx2.at[j]], add=True)
        return 0
    lax.fori_loop(0, PH1_ROWS, scat, 0)

    plsc.subcore_barrier()

    pltpu.sync_copy(counts, nodew)

    def recip(i, _):
        c = nodew[pl.ds(i * 16, 16)]
        nodew[pl.ds(i * 16, 16)] = 1.0 / jnp.maximum(c, 1.0)
        return 0
    lax.fori_loop(0, NP // 16, recip, 0)

    pltpu.sync_copy(batch_hbm, batchv)
    pltpu.sync_copy(dstf_hbm.at[pl.ds(wid * PER_TILE, PER_TILE)], idxf)

    def gath(i, _):
        ix = idxf[pl.ds(i * 16, 16)]
        wout[pl.ds(i * 16, 16)] = plsc.load_gather(nodew, [ix])
        gout[pl.ds(i * 16, 16)] = plsc.load_gather(batchv, [ix]).astype(jnp.float32)
        return 0
    lax.fori_loop(0, PER_TILE // 16, gath, 0)

    pltpu.sync_copy(wout, w_hbm.at[pl.ds(wid * PER_TILE, PER_TILE)])
    pltpu.sync_copy(gout, g_hbm.at[pl.ds(wid * PER_TILE, PER_TILE)])


def _sc_weights(dst2d, dstf, batchp):
    mesh = plsc.VectorSubcoreMesh(core_axis_name="c", subcore_axis_name="s",
                                  num_cores=NC, num_subcores=NS)
    f32 = jnp.float32
    return pl.kernel(
        _sc_body,
        out_type=(jax.ShapeDtypeStruct((EP,), f32),
                  jax.ShapeDtypeStruct((EP,), f32)),
        mesh=mesh,
        compiler_params=pltpu.CompilerParams(needs_layout_passes=False),
        scratch_types=(
            pltpu.VMEM((PH1_ROWS, 128), jnp.int32),
            pltpu.VMEM((PER_TILE,), jnp.int32),
            pltpu.VMEM((128,), f32),
            pltpu.VMEM((NP,), f32),
            pltpu.VMEM((NP,), jnp.int32),
            pltpu.VMEM((PER_TILE,), f32),
            pltpu.VMEM((PER_TILE,), f32),
            pltpu.VMEM_SHARED((NP,), f32),
        ),
    )(dst2d, dstf, batchp)


_C0 = 1.0 / math.sqrt(4.0 * math.pi)
_S15 = math.sqrt(15.0)
_S5 = math.sqrt(5.0)


def _silu(x):
    return x / (1.0 + jnp.exp(-x))


def _tc_body(ff, ev, w3, g3, wcat, b1s, b1i, w2s, w2i, b2p, cmt, out, acc):
    i = pl.program_id(0)

    @pl.when(i == 0)
    def _():
        acc[...] = jnp.zeros_like(acc)

    f = ff[...]
    v = ev[...]
    nrm = jnp.sqrt(jnp.sum(v * v, axis=1, keepdims=True))
    d = v / (nrm + 1e-12)
    x, y, z = d[:, 0:1], d[:, 1:2], d[:, 2:3]
    sh0 = (_S15 * _C0) * x * z
    sh1 = (_S15 * _C0) * x * y
    sh2 = _S5 * _C0 * (y * y - 0.5 * (x * x + z * z))
    sh3 = (_S15 * _C0) * y * z
    sh4 = (0.5 * _S15 * _C0) * (z * z - x * x)

    pre = jnp.dot(f, wcat[...], preferred_element_type=jnp.float32,
                  precision=lax.Precision.HIGHEST)
    pre_s = pre[:, :H] + b1s[...]
    pre_i = pre[:, H:]

    b2s = b2p[0, 0]
    b2i = b2p[0, 1]
    hs = _silu(pre_s)
    es = jnp.sum(hs * w2s[...], axis=1, keepdims=True) + b2s

    cols = [es, jnp.zeros((BLK, 3), jnp.float32)]
    for sh in (sh0, sh1, sh2, sh3, sh4):
        hm = _silu(sh * pre_i + b1i[...])
        cols.append(jnp.sum(hm * w2i[...], axis=1, keepdims=True) + b2i)
    cols.append(jnp.zeros((BLK, 16 - 9), jnp.float32))
    vals = jnp.concatenate(cols, axis=1)

    g = g3[...].reshape(1, BLK).astype(jnp.int32)
    w = w3[...].reshape(1, BLK)
    bins = lax.broadcasted_iota(jnp.int32, (NBP, BLK), 0)
    wone = jnp.where(bins == g, w, 0.0)
    acc[...] += jnp.dot(wone, vals, preferred_element_type=jnp.float32,
                        precision=lax.Precision.HIGHEST)

    @pl.when(i == GRID - 1)
    def _():
        out[...] = jnp.dot(acc[...], cmt[...],
                           preferred_element_type=jnp.float32,
                           precision=lax.Precision.HIGHEST)


def _tc_call(ff, ev, w3, g3, wcat, b1s, b1i, w2s, w2i, b2p, cmt):
    f32 = jnp.float32
    return pl.pallas_call(
        _tc_body,
        grid=(GRID,),
        in_specs=[
            pl.BlockSpec((BLK, H), lambda i: (i, 0)),
            pl.BlockSpec((BLK, 3), lambda i: (i, 0)),
            pl.BlockSpec((1, 1, BLK), lambda i: (i, 0, 0)),
            pl.BlockSpec((1, 1, BLK), lambda i: (i, 0, 0)),
            pl.BlockSpec((H, 2 * H), lambda i: (0, 0)),
            pl.BlockSpec((1, H), lambda i: (0, 0)),
            pl.BlockSpec((1, H), lambda i: (0, 0)),
            pl.BlockSpec((1, H), lambda i: (0, 0)),
            pl.BlockSpec((1, H), lambda i: (0, 0)),
            pl.BlockSpec((1, H), lambda i: (0, 0)),
            pl.BlockSpec((16, 16), lambda i: (0, 0)),
        ],
        out_specs=pl.BlockSpec((NBP, 16), lambda i: (0, 0)),
        out_shape=jax.ShapeDtypeStruct((NBP, 16), f32),
        scratch_shapes=[pltpu.VMEM((NBP, 16), f32)],
    )(ff, ev, w3, g3, wcat, b1s, b1i, w2s, w2i, b2p, cmt)


def kernel(force_features, edge_vectors, scalar_w1, scalar_b1, scalar_w2,
           scalar_b2, irrep2_w1, irrep2_b1, irrep2_w2, irrep2_b2,
           edge_index_dst, batch_idx):
    i32, f32 = jnp.int32, jnp.float32
    dstp = jnp.concatenate(
        [edge_index_dst.astype(i32), jnp.full((EP - E,), SENT, i32)])
    dst2d = dstp.reshape(ROWS, 128)
    batchp = jnp.concatenate([batch_idx.astype(i32), jnp.zeros((NP - NN,), i32)])

    w_e, g_e = _sc_weights(dst2d, dstp, batchp)

    w3 = w_e[:E].reshape(GRID, 1, BLK)
    g3 = g_e[:E].reshape(GRID, 1, BLK)
    wcat = jnp.concatenate([scalar_w1.T, irrep2_w1.T], axis=1)
    b1s = scalar_b1.reshape(1, H)
    b1i = irrep2_b1.reshape(1, H)
    w2s = scalar_w2.reshape(1, H)
    w2i = irrep2_w2.reshape(1, H)
    b2p = jnp.zeros((1, H), f32).at[0, 0].set(scalar_b2[0]).at[0, 1].set(irrep2_b2[0])
    cmt = jnp.zeros((16, 16), f32).at[:9, :9].set(_change_basis().T)

    out = _tc_call(force_features, edge_vectors, w3, g3, wcat,
                   b1s, b1i, w2s, w2i, b2p, cmt)
    return out[:NB, :9].reshape(NB, 3, 3)


# speedup vs baseline: 6.0291x; 2.3579x over previous
import functools
import math

import jax
import jax.numpy as jnp
from jax import lax
from jax.experimental import pallas as pl
from jax.experimental.pallas import tpu as pltpu
from jax.experimental.pallas import tpu_sc as plsc

E = 160000
H = 128
NN = 10000
NB = 100

EP = 163840
ROWS = EP // 128
NP = 10016
SENT = 10000

NC, NS = 2, 16
PH1_ROWS = ROWS // NS
PER_TILE = EP // (NC * NS)

BLK = 1600
GRID = E // BLK
NBP = 104


def _change_basis():
    m = [
        [3 ** -0.5, 0, 0, 0, 3 ** -0.5, 0, 0, 0, 3 ** -0.5],
        [0, 0, 0, 0, 0, 2 ** -0.5, 0, -(2 ** -0.5), 0],
        [0, 0, -(2 ** -0.5), 0, 0, 0, 2 ** -0.5, 0, 0],
        [0, 2 ** -0.5, 0, -(2 ** -0.5), 0, 0, 0, 0, 0],
        [0, 0, 0.5 ** 0.5, 0, 0, 0, 0.5 ** 0.5, 0, 0],
        [0, 0, 0, 0, 0, 2 ** -0.5, 0, 2 ** -0.5, 0],
        [-(6 ** -0.5), 0, 0, 0, 2 * 6 ** -0.5, 0, 0, 0, -(6 ** -0.5)],
        [0, 2 ** -0.5, 0, 2 ** -0.5, 0, 0, 0, 0, 0],
        [-(2 ** -0.5), 0, 0, 0, 0, 0, 0, 0, 2 ** -0.5],
    ]
    return jnp.asarray(m, dtype=jnp.float32).T


def _sc_body(dst2d_hbm, dstf_hbm, batch_hbm, w_hbm, g_hbm,
             idx2, idxf, ones, nodew, batchv, wout, gout, counts):
    cid = lax.axis_index("c")
    sid = lax.axis_index("s")
    wid = sid * NC + cid

    def fill16(ref, val, n16):
        def bdy(i, _):
            ref[pl.ds(i * 16, 16)] = jnp.full((16,), val, ref.dtype)
            return 0
        lax.fori_loop(0, n16, bdy, 0)

    fill16(ones, 1.0, 128 // 16)

    @pl.when(sid == 0)
    def _():
        fill16(nodew, 0.0, NP // 16)
        pltpu.sync_copy(nodew, counts)

    plsc.subcore_barrier()

    pltpu.sync_copy(dst2d_hbm.at[pl.ds(sid * PH1_ROWS, PH1_ROWS)], idx2)

    def scat(j, _):
        pltpu.sync_copy(ones, counts.at[idx2.at[j]], add=True)
        return 0
    lax.fori_loop(0, PH1_ROWS, scat, 0)

    plsc.subcore_barrier()

    pltpu.sync_copy(counts, nodew)

    def recip(i, _):
        c = nodew[pl.ds(i * 16, 16)]
        nodew[pl.ds(i * 16, 16)] = 1.0 / jnp.maximum(c, 1.0)
        return 0
    lax.fori_loop(0, NP // 16, recip, 0)

    pltpu.sync_copy(batch_hbm, batchv)
    pltpu.sync_copy(dstf_hbm.at[pl.ds(wid * PER_TILE, PER_TILE)], idxf)

    def gath(i, _):
        ix = idxf[pl.ds(i * 16, 16)]
        wout[pl.ds(i * 16, 16)] = plsc.load_gather(nodew, [ix])
        gout[pl.ds(i * 16, 16)] = plsc.load_gather(batchv, [ix]).astype(jnp.float32)
        return 0
    lax.fori_loop(0, PER_TILE // 16, gath, 0)

    pltpu.sync_copy(wout, w_hbm.at[pl.ds(wid * PER_TILE, PER_TILE)])
    pltpu.sync_copy(gout, g_hbm.at[pl.ds(wid * PER_TILE, PER_TILE)])


def _sc_weights(dst2d, dstf, batchp):
    mesh = plsc.VectorSubcoreMesh(core_axis_name="c", subcore_axis_name="s",
                                  num_cores=NC, num_subcores=NS)
    f32 = jnp.float32
    return pl.kernel(
        _sc_body,
        out_type=(jax.ShapeDtypeStruct((EP,), f32),
                  jax.ShapeDtypeStruct((EP,), f32)),
        mesh=mesh,
        compiler_params=pltpu.CompilerParams(needs_layout_passes=False),
        scratch_types=(
            pltpu.VMEM((PH1_ROWS, 128), jnp.int32),
            pltpu.VMEM((PER_TILE,), jnp.int32),
            pltpu.VMEM((128,), f32),
            pltpu.VMEM((NP,), f32),
            pltpu.VMEM((NP,), jnp.int32),
            pltpu.VMEM((PER_TILE,), f32),
            pltpu.VMEM((PER_TILE,), f32),
            pltpu.VMEM_SHARED((NP,), f32),
        ),
    )(dst2d, dstf, batchp)


_C0 = 1.0 / math.sqrt(4.0 * math.pi)
_S15 = math.sqrt(15.0)
_S5 = math.sqrt(5.0)


def _silu(x):
    return x / (1.0 + jnp.exp(-x))


def _tc_body(ff, ev, w3, g3, wcat, b1s, b1i, w2blk, b2row, cmt, out, acc):
    i = pl.program_id(0)

    @pl.when(i == 0)
    def _():
        acc[...] = jnp.zeros_like(acc)

    f = ff[...]
    vt = ev[...].reshape(3, BLK)
    x, y, z = vt[0:1, :], vt[1:2, :], vt[2:3, :]
    inv = 1.0 / (jnp.sqrt(x * x + y * y + z * z) + 1e-12)
    x, y, z = x * inv, y * inv, z * inv
    sh_rows = jnp.concatenate([
        (_S15 * _C0) * x * z,
        (_S15 * _C0) * x * y,
        (_S5 * _C0) * (y * y - 0.5 * (x * x + z * z)),
        (_S15 * _C0) * y * z,
        (0.5 * _S15 * _C0) * (z * z - x * x),
        jnp.zeros((3, BLK), jnp.float32),
    ], axis=0)
    shc = sh_rows.T
    sh0, sh1, sh2 = shc[:, 0:1], shc[:, 1:2], shc[:, 2:3]
    sh3, sh4 = shc[:, 3:4], shc[:, 4:5]

    pre = jnp.dot(f, wcat[...], preferred_element_type=jnp.float32)
    pre_s = pre[:, :H] + b1s[...]
    pre_i = pre[:, H:]

    w2 = w2blk[...]
    vals = jnp.dot(_silu(pre_s), w2[0:H, :],
                   preferred_element_type=jnp.float32)
    for m, sh in enumerate((sh0, sh1, sh2, sh3, sh4)):
        hm = _silu(sh * pre_i + b1i[...])
        vals += jnp.dot(hm, w2[(m + 1) * H:(m + 2) * H, :],
                        preferred_element_type=jnp.float32)
    vals += b2row[...]

    g = g3[...].reshape(1, BLK).astype(jnp.int32)
    w = w3[...].reshape(1, BLK)
    bins = lax.broadcasted_iota(jnp.int32, (NBP, BLK), 0)
    wone = jnp.where(bins == g, w, 0.0)
    acc[...] += jnp.dot(wone, vals, preferred_element_type=jnp.float32)

    @pl.when(i == GRID - 1)
    def _():
        out[...] = jnp.dot(acc[...], cmt[...],
                           preferred_element_type=jnp.float32,
                           precision=lax.Precision.HIGHEST)


def _tc_call(ff, ev, w3, g3, wcat, b1s, b1i, w2blk, b2row, cmt):
    f32 = jnp.float32
    return pl.pallas_call(
        _tc_body,
        grid=(GRID,),
        in_specs=[
            pl.BlockSpec((BLK, H), lambda i: (i, 0)),
            pl.BlockSpec((1, 3, BLK), lambda i: (i, 0, 0)),
            pl.BlockSpec((1, 1, BLK), lambda i: (i, 0, 0)),
            pl.BlockSpec((1, 1, BLK), lambda i: (i, 0, 0)),
            pl.BlockSpec((H, 2 * H), lambda i: (0, 0)),
            pl.BlockSpec((1, H), lambda i: (0, 0)),
            pl.BlockSpec((1, H), lambda i: (0, 0)),
            pl.BlockSpec((6 * H, 16), lambda i: (0, 0)),
            pl.BlockSpec((1, 16), lambda i: (0, 0)),
            pl.BlockSpec((16, 16), lambda i: (0, 0)),
        ],
        out_specs=pl.BlockSpec((NBP, 16), lambda i: (0, 0)),
        out_shape=jax.ShapeDtypeStruct((NBP, 16), f32),
        scratch_shapes=[pltpu.VMEM((NBP, 16), f32)],
    )(ff, ev, w3, g3, wcat, b1s, b1i, w2blk, b2row, cmt)


def kernel(force_features, edge_vectors, scalar_w1, scalar_b1, scalar_w2,
           scalar_b2, irrep2_w1, irrep2_b1, irrep2_w2, irrep2_b2,
           edge_index_dst, batch_idx):
    i32, f32 = jnp.int32, jnp.float32
    dstp = jnp.concatenate(
        [edge_index_dst.astype(i32), jnp.full((EP - E,), SENT, i32)])
    dst2d = dstp.reshape(ROWS, 128)
    batchp = jnp.concatenate([batch_idx.astype(i32), jnp.zeros((NP - NN,), i32)])

    w_e, g_e = _sc_weights(dst2d, dstp, batchp)

    w3 = w_e[:E].reshape(GRID, 1, BLK)
    g3 = g_e[:E].reshape(GRID, 1, BLK)
    wcat = jnp.concatenate([scalar_w1.T, irrep2_w1.T], axis=1)
    b1s = scalar_b1.reshape(1, H)
    b1i = irrep2_b1.reshape(1, H)
    w2blk = jnp.zeros((6 * H, 16), f32)
    w2blk = w2blk.at[0:H, 0].set(scalar_w2.reshape(H))
    for m in range(5):
        w2blk = w2blk.at[(m + 1) * H:(m + 2) * H, 4 + m].set(irrep2_w2.reshape(H))
    b2row = jnp.zeros((1, 16), f32)
    b2row = b2row.at[0, 0].set(scalar_b2[0])
    b2row = b2row.at[0, 4:9].set(irrep2_b2[0])
    cmt = jnp.zeros((16, 16), f32).at[:9, :9].set(_change_basis().T)

    out = _tc_call(force_features,
                   edge_vectors.reshape(GRID, BLK, 3).transpose(0, 2, 1),
                   w3, g3, wcat,
                   b1s, b1i, w2blk, b2row, cmt)
    return out[:NB, :9].reshape(NB, 3, 3)
